# baseline (speedup 1.0000x reference)
import functools

import jax
import jax.numpy as jnp
from jax import lax
from jax.experimental import pallas as pl
from jax.experimental.pallas import tpu as pltpu
from jax.experimental.pallas import tpu_sc as plsc

_NUM_CORES = 2
_NUM_SUBCORES = 16
_NUM_WORKERS = _NUM_CORES * _NUM_SUBCORES
_NBUF = 3
_LOOKAHEAD = 2


def _gather_body(n_chunks, chunk, width, table_hbm, idx_hbm, out_hbm,
                 idx_v, jdx_v, pairs_v, comp_v, gsem, wsem):
    wid = lax.axis_index("s") * _NUM_CORES + lax.axis_index("c")
    base = wid * (n_chunks * chunk)

    def stage_idx(j, b):
        pltpu.sync_copy(idx_hbm.at[pl.ds(base + j * chunk, chunk)],
                        idx_v.at[b])
        for v in range(chunk // 16):
            sl = pl.ds(v * 16, 16)
            jdx_v[b, sl] = lax.shift_right_logical(idx_v[b, sl], 1)

    def fire_gather(b):
        pltpu.async_copy(table_hbm.at[jdx_v.at[b]], pairs_v.at[b],
                         gsem.at[b])

    def wait_gather(b):
        pltpu.make_async_copy(table_hbm.at[jdx_v.at[b]], pairs_v.at[b],
                              gsem.at[b]).wait()

    def select(b):
        pairs_ref = pairs_v.at[b]
        comp_ref = comp_v.at[b]

        def group(g, carry):
            k0 = g * 16
            rows = k0 + lax.iota(jnp.int32, 16)
            hoff = (idx_v[b, pl.ds(k0, 16)] & 1) * width
            for p in range(width):
                col = hoff + p
                v = plsc.load_gather(pairs_ref, [rows, col])
                plsc.store_scatter(comp_ref,
                                   [rows, jnp.full((16,), p, jnp.int32)], v)
            return carry
        lax.fori_loop(0, chunk // 16, group, 0)

    def fire_wb(j, b):
        pltpu.async_copy(comp_v.at[b], out_hbm.at[pl.ds(base + j * chunk,
                                                        chunk)], wsem.at[b])

    def wait_wb(j, b):
        pltpu.make_async_copy(comp_v.at[b],
                              out_hbm.at[pl.ds(base + j * chunk, chunk)],
                              wsem.at[b]).wait()

    for j in range(_LOOKAHEAD):
        stage_idx(j, j)
        fire_gather(j)

    def chunk_step(j, b, bn, *, do_fire, do_wait_wb):
        wait_gather(b)
        if do_fire:
            stage_idx(j + _LOOKAHEAD, bn)
            fire_gather(bn)
        if do_wait_wb:
            wait_wb(j - _NBUF, b)
        select(b)
        fire_wb(j, b)

    for j in range(_NBUF):
        chunk_step(j, j % _NBUF, (j + _LOOKAHEAD) % _NBUF,
                   do_fire=True, do_wait_wb=False)

    def step(jo, carry):
        j0 = jo * _NBUF
        for b in range(_NBUF):
            chunk_step(j0 + b, b, (b + _LOOKAHEAD) % _NBUF,
                       do_fire=True, do_wait_wb=True)
        return carry

    lax.fori_loop(1, (n_chunks - _LOOKAHEAD) // _NBUF, step, 0)

    for j in range(n_chunks - _LOOKAHEAD, n_chunks):
        b = j % _NBUF
        chunk_step(j, b, (b + _LOOKAHEAD) % _NBUF,
                   do_fire=False, do_wait_wb=True)
    for j in range(n_chunks - _NBUF, n_chunks):
        wait_wb(j, j % _NBUF)


@functools.partial(jax.jit, static_argnames=("n_rows", "chunk"))
def _sc_embedding_lookup(idx_flat, table_pairs, *, n_rows, chunk):
    pair_width = table_pairs.shape[1]
    width = pair_width // 2
    per_worker = n_rows // _NUM_WORKERS
    n_chunks = per_worker // chunk
    mesh = plsc.VectorSubcoreMesh(
        core_axis_name="c", subcore_axis_name="s",
        num_cores=_NUM_CORES, num_subcores=_NUM_SUBCORES)
    body = functools.partial(_gather_body, n_chunks, chunk, width)
    return pl.kernel(
        body,
        out_type=jax.ShapeDtypeStruct((n_rows, width), jnp.float32),
        mesh=mesh,
        scratch_types=[
            pltpu.VMEM((_NBUF, chunk), jnp.int32),
            pltpu.VMEM((_NBUF, chunk), jnp.int32),
            pltpu.VMEM((_NBUF, chunk, pair_width), jnp.float32),
            pltpu.VMEM((_NBUF, chunk, width), jnp.float32),
            pltpu.SemaphoreType.DMA((_NBUF,)),
            pltpu.SemaphoreType.DMA((_NBUF,)),
        ],
        compiler_params=pltpu.CompilerParams(use_tc_tiling_on_sc=True,
                                             needs_layout_passes=False),
    )(table_pairs, idx_flat)


def kernel(inputs, embeddings):
    n_rows = inputs.shape[0] * inputs.shape[1]
    width = embeddings.shape[1]
    idx_flat = jnp.reshape(inputs, (n_rows,)).astype(jnp.int32)
    table_pairs = jnp.reshape(embeddings,
                              (embeddings.shape[0] // 2, 2 * width))
    out = _sc_embedding_lookup(idx_flat, table_pairs, n_rows=n_rows,
                               chunk=128)
    return jnp.reshape(out, inputs.shape + (width,))


# speedup vs baseline: 1.1993x; 1.1993x over previous
import functools

import jax
import jax.numpy as jnp
from jax import lax
from jax.experimental import pallas as pl
from jax.experimental.pallas import tpu as pltpu
from jax.experimental.pallas import tpu_sc as plsc

_NUM_CORES = 2
_NUM_SUBCORES = 16
_NUM_WORKERS = _NUM_CORES * _NUM_SUBCORES
_NBUF = 4
_LOOKAHEAD = 2
_GCOLS = 512


def _transpose_body(n_cols, width, tt_hbm, tail_hbm, t128_hbm,
                    slab_v, tbuf_v, tails_v, sem):
    del sem
    wid = lax.axis_index("s") * _NUM_CORES + lax.axis_index("c")
    n_full = n_cols // _GCOLS
    per_tile = n_full // _NUM_WORKERS
    n_tail = n_cols - n_full * _GCOLS
    iotas = [q * 16 + lax.iota(jnp.int32, 16) for q in range(width // 16)]

    def do_group(g):
        c0 = g * _GCOLS
        pltpu.sync_copy(tt_hbm.at[:, pl.ds(c0, _GCOLS)], slab_v)

        def col(c, carry):
            cvec = jnp.zeros((16,), jnp.int32) + c
            for q in range(width // 16):
                v = plsc.load_gather(slab_v, [iotas[q], cvec])
                tbuf_v[c, pl.ds(q * 16, 16)] = v
            return carry

        lax.fori_loop(0, _GCOLS, col, 0)
        pltpu.sync_copy(tbuf_v, t128_hbm.at[pl.ds(c0, _GCOLS)])

    def step(j, carry):
        do_group(j * _NUM_WORKERS + wid)
        return carry

    lax.fori_loop(0, per_tile, step, 0)

    rem_full = n_full - per_tile * _NUM_WORKERS
    for r in range(rem_full):
        @pl.when(wid == r)
        def _():
            do_group(per_tile * _NUM_WORKERS + r)

    if n_tail:
        @pl.when(wid == rem_full)
        def _():
            pltpu.sync_copy(tail_hbm, tails_v)

            def trow(c, carry):
                for q in range(width // 16):
                    sl = pl.ds(q * 16, 16)
                    tbuf_v[c, sl] = tails_v[c, sl]
                return carry

            lax.fori_loop(0, n_tail, trow, 0)
            pltpu.sync_copy(tbuf_v.at[pl.ds(0, n_tail)],
                            t128_hbm.at[pl.ds(n_full * _GCOLS, n_tail)])


@jax.jit
def _sc_transpose_pad(tt, tail):
    width, n_cols = tt.shape
    mesh = plsc.VectorSubcoreMesh(
        core_axis_name="c", subcore_axis_name="s",
        num_cores=_NUM_CORES, num_subcores=_NUM_SUBCORES)
    body = functools.partial(_transpose_body, n_cols, width)
    return pl.kernel(
        body,
        out_type=jax.ShapeDtypeStruct((n_cols, 128), jnp.float32),
        mesh=mesh,
        scratch_types=[
            pltpu.VMEM((width, _GCOLS), jnp.float32),
            pltpu.VMEM((_GCOLS, 128), jnp.float32),
            pltpu.VMEM((tail.shape[0], width), jnp.float32),
            pltpu.SemaphoreType.DMA,
        ],
        compiler_params=pltpu.CompilerParams(use_tc_tiling_on_sc=True,
                                             needs_layout_passes=False),
    )(tt, tail)


def _gather_body(n_chunks, chunk, table_hbm, idx_hbm, out_hbm,
                 idx_v, rows_v, gsem, wsem):
    wid = lax.axis_index("s") * _NUM_CORES + lax.axis_index("c")
    base = wid * (n_chunks * chunk)

    def load_idx(j, b):
        pltpu.sync_copy(idx_hbm.at[pl.ds(base + j * chunk, chunk)],
                        idx_v.at[b])

    def fire_gather(b):
        pltpu.async_copy(table_hbm.at[idx_v.at[b]], rows_v.at[b],
                         gsem.at[b])

    def wait_gather(b):
        pltpu.make_async_copy(table_hbm.at[idx_v.at[b]], rows_v.at[b],
                              gsem.at[b]).wait()

    def fire_wb(j, b):
        pltpu.async_copy(rows_v.at[b], out_hbm.at[pl.ds(base + j * chunk,
                                                        chunk)], wsem.at[b])

    def wait_wb(j, b):
        pltpu.make_async_copy(rows_v.at[b],
                              out_hbm.at[pl.ds(base + j * chunk, chunk)],
                              wsem.at[b]).wait()

    load_idx(0, 0)
    fire_gather(0)
    load_idx(1, 1)
    fire_gather(1)

    for j in range(4):
        b, bn = j % _NBUF, (j + 2) % _NBUF
        wait_gather(b)
        fire_wb(j, b)
        if j >= 2:
            wait_wb(j - 2, bn)
        load_idx(j + 2, bn)
        fire_gather(bn)

    def step(jo, carry):
        j0 = jo * _NBUF
        for b in range(_NBUF):
            j = j0 + b
            bn = (b + 2) % _NBUF
            wait_gather(b)
            fire_wb(j, b)
            wait_wb(j - 2, bn)
            load_idx(j + 2, bn)
            fire_gather(bn)
        return carry

    lax.fori_loop(1, n_chunks // _NBUF - 1, step, 0)

    for j in range(n_chunks - 4, n_chunks):
        b = j % _NBUF
        wait_gather(b)
        fire_wb(j, b)
        if j + 2 < n_chunks:
            bn = (b + 2) % _NBUF
            wait_wb(j - 2, bn)
            load_idx(j + 2, bn)
            fire_gather(bn)
    for j in range(n_chunks - 4, n_chunks):
        wait_wb(j, j % _NBUF)


@functools.partial(jax.jit, static_argnames=("n_rows", "chunk"))
def _sc_embedding_lookup(idx_flat, table, *, n_rows, chunk):
    width = table.shape[1]
    per_worker = n_rows // _NUM_WORKERS
    n_chunks = per_worker // chunk
    mesh = plsc.VectorSubcoreMesh(
        core_axis_name="c", subcore_axis_name="s",
        num_cores=_NUM_CORES, num_subcores=_NUM_SUBCORES)
    body = functools.partial(_gather_body, n_chunks, chunk)
    return pl.kernel(
        body,
        out_type=jax.ShapeDtypeStruct((n_rows, width), jnp.float32),
        mesh=mesh,
        scratch_types=[
            pltpu.VMEM((_NBUF, chunk), jnp.int32),
            pltpu.VMEM((_NBUF, chunk, width), jnp.float32),
            pltpu.SemaphoreType.DMA((_NBUF,)),
            pltpu.SemaphoreType.DMA((_NBUF,)),
        ],
        compiler_params=pltpu.CompilerParams(use_tc_tiling_on_sc=True),
    )(table, idx_flat)


def kernel(inputs, embeddings):
    n_rows = inputs.shape[0] * inputs.shape[1]
    width = embeddings.shape[1]
    idx_flat = jnp.reshape(inputs, (n_rows,)).astype(jnp.int32)
    n_full = embeddings.shape[0] // _GCOLS * _GCOLS
    table = _sc_transpose_pad(jnp.transpose(embeddings),
                              embeddings[n_full:, :])
    out = _sc_embedding_lookup(idx_flat, table, n_rows=n_rows, chunk=128)
    return jnp.reshape(out[:, :width], inputs.shape + (width,))


# speedup vs baseline: 1.2006x; 1.0011x over previous
import functools

import jax
import jax.numpy as jnp
from jax import lax
from jax.experimental import pallas as pl
from jax.experimental.pallas import tpu as pltpu
from jax.experimental.pallas import tpu_sc as plsc

_NUM_CORES = 2
_NUM_SUBCORES = 16
_NUM_WORKERS = _NUM_CORES * _NUM_SUBCORES
_NBUF = 4
_LOOKAHEAD = 2
_GCOLS = 512


def _transpose_body(n_cols, width, tt_hbm, tail_hbm, t128_hbm,
                    slab_v, tbuf_v, tails_v, sem):
    del sem
    wid = lax.axis_index("s") * _NUM_CORES + lax.axis_index("c")
    n_full = n_cols // _GCOLS
    per_tile = n_full // _NUM_WORKERS
    n_tail = n_cols - n_full * _GCOLS
    iotas = [q * 16 + lax.iota(jnp.int32, 16) for q in range(width // 16)]

    def do_group(g):
        c0 = g * _GCOLS
        pltpu.sync_copy(tt_hbm.at[:, pl.ds(c0, _GCOLS)], slab_v)

        def col(c, carry):
            cvec = jnp.zeros((16,), jnp.int32) + c
            for q in range(width // 16):
                v = plsc.load_gather(slab_v, [iotas[q], cvec])
                tbuf_v[c, pl.ds(q * 16, 16)] = v
            return carry

        lax.fori_loop(0, _GCOLS, col, 0, unroll=8)
        pltpu.sync_copy(tbuf_v, t128_hbm.at[pl.ds(c0, _GCOLS)])

    def step(j, carry):
        do_group(j * _NUM_WORKERS + wid)
        return carry

    lax.fori_loop(0, per_tile, step, 0)

    rem_full = n_full - per_tile * _NUM_WORKERS
    for r in range(rem_full):
        @pl.when(wid == r)
        def _():
            do_group(per_tile * _NUM_WORKERS + r)

    if n_tail:
        @pl.when(wid == rem_full)
        def _():
            pltpu.sync_copy(tail_hbm, tails_v)

            def trow(c, carry):
                for q in range(width // 16):
                    sl = pl.ds(q * 16, 16)
                    tbuf_v[c, sl] = tails_v[c, sl]
                return carry

            lax.fori_loop(0, n_tail, trow, 0)
            pltpu.sync_copy(tbuf_v.at[pl.ds(0, n_tail)],
                            t128_hbm.at[pl.ds(n_full * _GCOLS, n_tail)])


@jax.jit
def _sc_transpose_pad(tt, tail):
    width, n_cols = tt.shape
    mesh = plsc.VectorSubcoreMesh(
        core_axis_name="c", subcore_axis_name="s",
        num_cores=_NUM_CORES, num_subcores=_NUM_SUBCORES)
    body = functools.partial(_transpose_body, n_cols, width)
    return pl.kernel(
        body,
        out_type=jax.ShapeDtypeStruct((n_cols, 128), jnp.float32),
        mesh=mesh,
        scratch_types=[
            pltpu.VMEM((width, _GCOLS), jnp.float32),
            pltpu.VMEM((_GCOLS, 128), jnp.float32),
            pltpu.VMEM((tail.shape[0], width), jnp.float32),
            pltpu.SemaphoreType.DMA,
        ],
        compiler_params=pltpu.CompilerParams(use_tc_tiling_on_sc=True,
                                             needs_layout_passes=False),
    )(tt, tail)


def _gather_body(n_chunks, chunk, table_hbm, idx_hbm, out_hbm,
                 idx_v, rows_v, gsem, wsem):
    wid = lax.axis_index("s") * _NUM_CORES + lax.axis_index("c")
    base = wid * (n_chunks * chunk)

    def load_idx(j, b):
        pltpu.sync_copy(idx_hbm.at[pl.ds(base + j * chunk, chunk)],
                        idx_v.at[b])

    def fire_gather(b):
        pltpu.async_copy(table_hbm.at[idx_v.at[b]], rows_v.at[b],
                         gsem.at[b])

    def wait_gather(b):
        pltpu.make_async_copy(table_hbm.at[idx_v.at[b]], rows_v.at[b],
                              gsem.at[b]).wait()

    def fire_wb(j, b):
        pltpu.async_copy(rows_v.at[b], out_hbm.at[pl.ds(base + j * chunk,
                                                        chunk)], wsem.at[b])

    def wait_wb(j, b):
        pltpu.make_async_copy(rows_v.at[b],
                              out_hbm.at[pl.ds(base + j * chunk, chunk)],
                              wsem.at[b]).wait()

    load_idx(0, 0)
    fire_gather(0)
    load_idx(1, 1)
    fire_gather(1)

    for j in range(4):
        b, bn = j % _NBUF, (j + 2) % _NBUF
        wait_gather(b)
        fire_wb(j, b)
        if j >= 2:
            wait_wb(j - 2, bn)
        load_idx(j + 2, bn)
        fire_gather(bn)

    def step(jo, carry):
        j0 = jo * _NBUF
        for b in range(_NBUF):
            j = j0 + b
            bn = (b + 2) % _NBUF
            wait_gather(b)
            fire_wb(j, b)
            wait_wb(j - 2, bn)
            load_idx(j + 2, bn)
            fire_gather(bn)
        return carry

    lax.fori_loop(1, n_chunks // _NBUF - 1, step, 0)

    for j in range(n_chunks - 4, n_chunks):
        b = j % _NBUF
        wait_gather(b)
        fire_wb(j, b)
        if j + 2 < n_chunks:
            bn = (b + 2) % _NBUF
            wait_wb(j - 2, bn)
            load_idx(j + 2, bn)
            fire_gather(bn)
    for j in range(n_chunks - 4, n_chunks):
        wait_wb(j, j % _NBUF)


@functools.partial(jax.jit, static_argnames=("n_rows", "chunk"))
def _sc_embedding_lookup(idx_flat, table, *, n_rows, chunk):
    width = table.shape[1]
    per_worker = n_rows // _NUM_WORKERS
    n_chunks = per_worker // chunk
    mesh = plsc.VectorSubcoreMesh(
        core_axis_name="c", subcore_axis_name="s",
        num_cores=_NUM_CORES, num_subcores=_NUM_SUBCORES)
    body = functools.partial(_gather_body, n_chunks, chunk)
    return pl.kernel(
        body,
        out_type=jax.ShapeDtypeStruct((n_rows, width), jnp.float32),
        mesh=mesh,
        scratch_types=[
            pltpu.VMEM((_NBUF, chunk), jnp.int32),
            pltpu.VMEM((_NBUF, chunk, width), jnp.float32),
            pltpu.SemaphoreType.DMA((_NBUF,)),
            pltpu.SemaphoreType.DMA((_NBUF,)),
        ],
        compiler_params=pltpu.CompilerParams(use_tc_tiling_on_sc=True),
    )(table, idx_flat)


def kernel(inputs, embeddings):
    n_rows = inputs.shape[0] * inputs.shape[1]
    width = embeddings.shape[1]
    idx_flat = jnp.reshape(inputs, (n_rows,)).astype(jnp.int32)
    n_full = embeddings.shape[0] // _GCOLS * _GCOLS
    table = _sc_transpose_pad(jnp.transpose(embeddings),
                              embeddings[n_full:, :])
    out = _sc_embedding_lookup(idx_flat, table, n_rows=n_rows, chunk=128)
    return jnp.reshape(out[:, :width], inputs.shape + (width,))


# speedup vs baseline: 2.7016x; 2.2501x over previous
import functools

import jax
import jax.numpy as jnp
from jax import lax
from jax.experimental import pallas as pl
from jax.experimental.pallas import tpu as pltpu
from jax.experimental.pallas import tpu_sc as plsc

_NUM_CORES = 2
_NUM_SUBCORES = 16
_NUM_WORKERS = _NUM_CORES * _NUM_SUBCORES
_NBUF = 4


def _gather_body(n_chunks, chunk, table_hbm, idx_hbm, out_hbm,
                 idx_v, rows_v, gsem, wsem):
    wid = lax.axis_index("s") * _NUM_CORES + lax.axis_index("c")
    base = wid * (n_chunks * chunk)

    def load_idx(j, b):
        pltpu.sync_copy(idx_hbm.at[pl.ds(base + j * chunk, chunk)],
                        idx_v.at[b])

    def fire_gather(b):
        pltpu.async_copy(table_hbm.at[idx_v.at[b]], rows_v.at[b],
                         gsem.at[b])

    def wait_gather(b):
        pltpu.make_async_copy(table_hbm.at[idx_v.at[b]], rows_v.at[b],
                              gsem.at[b]).wait()

    def fire_wb(j, b):
        pltpu.async_copy(rows_v.at[b], out_hbm.at[pl.ds(base + j * chunk,
                                                        chunk)], wsem.at[b])

    def wait_wb(j, b):
        pltpu.make_async_copy(rows_v.at[b],
                              out_hbm.at[pl.ds(base + j * chunk, chunk)],
                              wsem.at[b]).wait()

    load_idx(0, 0)
    fire_gather(0)
    load_idx(1, 1)
    fire_gather(1)

    for j in range(4):
        b, bn = j % _NBUF, (j + 2) % _NBUF
        wait_gather(b)
        fire_wb(j, b)
        if j >= 2:
            wait_wb(j - 2, bn)
        load_idx(j + 2, bn)
        fire_gather(bn)

    def step(jo, carry):
        j0 = jo * _NBUF
        for b in range(_NBUF):
            j = j0 + b
            bn = (b + 2) % _NBUF
            wait_gather(b)
            fire_wb(j, b)
            wait_wb(j - 2, bn)
            load_idx(j + 2, bn)
            fire_gather(bn)
        return carry

    lax.fori_loop(1, n_chunks // _NBUF - 1, step, 0)

    for j in range(n_chunks - 4, n_chunks):
        b = j % _NBUF
        wait_gather(b)
        fire_wb(j, b)
        if j + 2 < n_chunks:
            bn = (b + 2) % _NBUF
            wait_wb(j - 2, bn)
            load_idx(j + 2, bn)
            fire_gather(bn)
    for j in range(n_chunks - 4, n_chunks):
        wait_wb(j, j % _NBUF)


@functools.partial(jax.jit, static_argnames=("n_rows", "chunk"))
def _sc_embedding_lookup(idx_flat, table, *, n_rows, chunk):
    width = table.shape[1]
    per_worker = n_rows // _NUM_WORKERS
    n_chunks = per_worker // chunk
    mesh = plsc.VectorSubcoreMesh(
        core_axis_name="c", subcore_axis_name="s",
        num_cores=_NUM_CORES, num_subcores=_NUM_SUBCORES)
    body = functools.partial(_gather_body, n_chunks, chunk)
    return pl.kernel(
        body,
        out_type=jax.ShapeDtypeStruct((n_rows, width), jnp.float32),
        mesh=mesh,
        scratch_types=[
            pltpu.VMEM((_NBUF, chunk), jnp.int32),
            pltpu.VMEM((_NBUF, chunk, width), jnp.float32),
            pltpu.SemaphoreType.DMA((_NBUF,)),
            pltpu.SemaphoreType.DMA((_NBUF,)),
        ],
        compiler_params=pltpu.CompilerParams(use_tc_tiling_on_sc=True),
    )(table, idx_flat)


def kernel(inputs, embeddings):
    n_rows = inputs.shape[0] * inputs.shape[1]
    width = embeddings.shape[1]
    idx_flat = jnp.reshape(inputs, (n_rows,)).astype(jnp.int32)
    table = jnp.pad(embeddings, ((0, 0), (0, 128 - width)))
    out = _sc_embedding_lookup(idx_flat, table, n_rows=n_rows, chunk=128)
    return jnp.reshape(out[:, :width], inputs.shape + (width,))


# speedup vs baseline: 2.9633x; 1.0969x over previous
import functools

import jax
import jax.numpy as jnp
from jax import lax
from jax.experimental import pallas as pl
from jax.experimental.pallas import tpu as pltpu
from jax.experimental.pallas import tpu_sc as plsc

_NUM_CORES = 2
_NUM_SUBCORES = 16
_NUM_WORKERS = _NUM_CORES * _NUM_SUBCORES
_NBUF = 4


def _gather_body(n_chunks, chunk, table_hbm, idx_hbm, out_hbm,
                 idx_v, rows_v, gsem, wsem):
    wid = lax.axis_index("s") * _NUM_CORES + lax.axis_index("c")
    base = wid * (n_chunks * chunk)

    def load_idx(j, b):
        pltpu.sync_copy(idx_hbm.at[pl.ds(base + j * chunk, chunk)],
                        idx_v.at[b])

    def fire_gather(b):
        pltpu.async_copy(table_hbm.at[idx_v.at[b]], rows_v.at[b],
                         gsem.at[b])

    def wait_gather(b):
        pltpu.make_async_copy(table_hbm.at[idx_v.at[b]], rows_v.at[b],
                              gsem.at[b]).wait()

    def fire_wb(j, b):
        pltpu.async_copy(rows_v.at[b],
                         out_hbm.at[pl.ds(base + j * chunk, chunk),
                                    pl.ds(0, rows_v.shape[2])], wsem.at[b])

    def wait_wb(j, b):
        pltpu.make_async_copy(rows_v.at[b],
                              out_hbm.at[pl.ds(base + j * chunk, chunk),
                                         pl.ds(0, rows_v.shape[2])],
                              wsem.at[b]).wait()

    load_idx(0, 0)
    fire_gather(0)
    load_idx(1, 1)
    fire_gather(1)

    for j in range(4):
        b, bn = j % _NBUF, (j + 2) % _NBUF
        wait_gather(b)
        fire_wb(j, b)
        if j >= 2:
            wait_wb(j - 2, bn)
        load_idx(j + 2, bn)
        fire_gather(bn)

    def step(jo, carry):
        j0 = jo * _NBUF
        for b in range(_NBUF):
            j = j0 + b
            bn = (b + 2) % _NBUF
            wait_gather(b)
            fire_wb(j, b)
            wait_wb(j - 2, bn)
            load_idx(j + 2, bn)
            fire_gather(bn)
        return carry

    lax.fori_loop(1, n_chunks // _NBUF - 1, step, 0)

    for j in range(n_chunks - 4, n_chunks):
        b = j % _NBUF
        wait_gather(b)
        fire_wb(j, b)
        if j + 2 < n_chunks:
            bn = (b + 2) % _NBUF
            wait_wb(j - 2, bn)
            load_idx(j + 2, bn)
            fire_gather(bn)
    for j in range(n_chunks - 4, n_chunks):
        wait_wb(j, j % _NBUF)


@functools.partial(jax.jit, static_argnames=("n_rows", "chunk"))
def _sc_embedding_lookup(idx_flat, table, *, n_rows, chunk):
    width = table.shape[1]
    per_worker = n_rows // _NUM_WORKERS
    n_chunks = per_worker // chunk
    mesh = plsc.VectorSubcoreMesh(
        core_axis_name="c", subcore_axis_name="s",
        num_cores=_NUM_CORES, num_subcores=_NUM_SUBCORES)
    body = functools.partial(_gather_body, n_chunks, chunk)
    return pl.kernel(
        body,
        out_type=jax.ShapeDtypeStruct((n_rows, 128), jnp.float32),
        mesh=mesh,
        scratch_types=[
            pltpu.VMEM((_NBUF, chunk), jnp.int32),
            pltpu.VMEM((_NBUF, chunk, width), jnp.float32),
            pltpu.SemaphoreType.DMA((_NBUF,)),
            pltpu.SemaphoreType.DMA((_NBUF,)),
        ],
        compiler_params=pltpu.CompilerParams(use_tc_tiling_on_sc=False),
    )(table, idx_flat)


def kernel(inputs, embeddings):
    n_rows = inputs.shape[0] * inputs.shape[1]
    width = embeddings.shape[1]
    idx_flat = jnp.reshape(inputs, (n_rows,)).astype(jnp.int32)
    out = _sc_embedding_lookup(idx_flat, embeddings, n_rows=n_rows,
                               chunk=400)
    return jnp.reshape(out[:, :width], inputs.shape + (width,))
